# baseline (speedup 1.0000x reference)
import jax
import jax.numpy as jnp
from jax import lax
from jax.experimental import pallas as pl
from jax.experimental.pallas import tpu as pltpu, tpu_sc as plsc

EMBED = 768
NEXP = 8
K = 384
INTER_S = 1536
T = 2048
TB = 256
NW = 32
TOK_PER_W = T // NW
CHT = 8
NCHUNK = TOK_PER_W // CHT
NV = EMBED // 16
KV = K // 16
MASK31 = 0x7FFFFFFF

_CONTRACT_MINOR = (((1,), (1,)), ((), ()))


def _dotT(a, b):
    return lax.dot_general(a, b, _CONTRACT_MINOR,
                           preferred_element_type=jnp.float32)


def _pe_body(pos_ref, Wp_ref, bp_ref, pe_ref):
    logits = _dotT(pos_ref[...], Wp_ref[...]) + bp_ref[...]
    z = logits - jnp.max(logits, axis=-1, keepdims=True)
    e = jnp.exp(z)
    pe_ref[...] = e / jnp.sum(e, axis=-1, keepdims=True)


def _gate_body(h_ref, pe_ref, Wg_ref, bg_ref, gate_ref):
    h = h_ref[...]
    for e in range(NEXP):
        gate_ref[e] = _dotT(h * pe_ref[e], Wg_ref[...]) + bg_ref[...]


def _shared_body(h_ref, Wgs_ref, Wus_ref, Wds_ref, y0_ref):
    h = h_ref[...]
    m = jax.nn.silu(_dotT(h, Wgs_ref[...])) * _dotT(h, Wus_ref[...])
    y0_ref[...] = _dotT(m, Wds_ref[...])


def _post_body(xe_ref, y0_ref, Wge_ref, Wue_ref, Wde_ref,
               lng_ref, lnb_ref, W1_ref, b1_ref, W2_ref, b2_ref, o_ref):
    y = y0_ref[...]
    for e in range(NEXP):
        x = xe_ref[e]
        a = _dotT(x, Wge_ref[e])
        b = _dotT(x, Wue_ref[e])
        y = y + _dotT(jax.nn.silu(a) * b, Wde_ref[e])
    mean = jnp.mean(y, axis=-1, keepdims=True)
    var = jnp.mean((y - mean) ** 2, axis=-1, keepdims=True)
    y = (y - mean) * lax.rsqrt(var + 1e-6) * lng_ref[...] + lnb_ref[...]
    z = jax.nn.silu(_dotT(y, W1_ref[...]) + b1_ref[...])
    o_ref[...] = _dotT(z, W2_ref[...]) + b2_ref[...]


def _desc_key(f):
    u = plsc.bitcast(f, jnp.int32)
    s = lax.shift_right_arithmetic(u, 31)
    return jnp.bitwise_xor(u, jnp.bitwise_and(jnp.bitwise_not(s), MASK31))


def _inv_desc_key(k):
    s = lax.shift_right_arithmetic(k, 31)
    u = jnp.bitwise_xor(k, jnp.bitwise_and(jnp.bitwise_not(s), MASK31))
    return plsc.bitcast(u, jnp.float32)


def _digit(k, p):
    if p:
        k = lax.shift_right_logical(k, 8 * p)
    return jnp.bitwise_and(k, 255)


def _sc_body(gate_hbm, h_hbm, xe_hbm,
             hbuf, gbuf, xbuf, kA, kB, pA, pB, ebuf, hist, offs, sbuf):
    wid = lax.axis_index("s") * 2 + lax.axis_index("c")
    zero16 = jnp.zeros((16,), jnp.int32)
    full15 = jnp.full((16,), 15, jnp.int32)

    def radix_pass(p, r):
        ksrc = kA if p % 2 == 0 else kB
        kdst = kB if p % 2 == 0 else kA
        pdst = pB if p % 2 == 0 else pA
        if p == 0:
            def psrc(sl):
                return hbuf[r, sl]
        elif p % 2 == 1:
            def psrc(sl):
                return pB[sl]
        else:
            def psrc(sl):
                return pA[sl]
        for v in range(16):
            hist[pl.ds(v * 16, 16)] = zero16
        for v in range(NV):
            d = _digit(ksrc[pl.ds(v * 16, 16)], p)
            cnt, lastm = plsc.scan_count(d)
            plsc.addupdate_scatter(hist, [d], cnt, mask=lastm)
        run = jnp.int32(0)
        for v in range(16):
            hv = hist[pl.ds(v * 16, 16)]
            inc = plsc.cumsum(hv)
            offs[pl.ds(v * 16, 16)] = inc - hv + run
            run = run + jnp.sum(hv)
        for v in range(NV):
            sl = pl.ds(v * 16, 16)
            k = ksrc[sl]
            pay = psrc(sl)
            d = _digit(k, p)
            cnt, lastm = plsc.scan_count(d)
            base = plsc.load_gather(offs, [d])
            pos = base + cnt - 1
            plsc.store_scatter(kdst, [pos], k)
            plsc.store_scatter(pdst, [pos], pay)
            plsc.addupdate_scatter(offs, [d], cnt, mask=lastm)

    def row_body(r, carry):
        for v in range(NV):
            sl = pl.ds(v * 16, 16)
            kA[sl] = _desc_key(gbuf[r, sl])
        for p in range(4):
            radix_pass(p, r)
        acc = jnp.zeros((16,), jnp.float32)
        v0 = _inv_desc_key(kA[pl.ds(0, 16)])
        sbuf[...] = plsc.cummax(v0)
        mv = plsc.load_gather(sbuf, [full15])
        for j in range(KV):
            sl = pl.ds(j * 16, 16)
            ej = jnp.exp(_inv_desc_key(kA[sl]) - mv)
            ebuf[sl] = ej
            acc = acc + ej
        sbuf[...] = plsc.cumsum(acc)
        tot = plsc.load_gather(sbuf, [full15])
        winv = 1.0 / tot
        for j in range(KV):
            sl = pl.ds(j * 16, 16)
            xbuf[r, sl] = ebuf[sl] * winv * pA[sl]
        return carry

    def exp_body(e, carry, t0):
        pltpu.sync_copy(gate_hbm.at[e, pl.ds(t0, CHT)], gbuf)
        lax.fori_loop(0, CHT, row_body, 0)
        pltpu.sync_copy(xbuf, xe_hbm.at[e, pl.ds(t0, CHT)])
        return carry

    def chunk_body(ci, carry):
        t0 = wid * TOK_PER_W + ci * CHT
        pltpu.sync_copy(h_hbm.at[pl.ds(t0, CHT)], hbuf)
        lax.fori_loop(0, NEXP, lambda e, c: exp_body(e, c, t0), 0)
        return carry

    lax.fori_loop(0, NCHUNK, chunk_body, 0)


def _sc_topk(gate_all, h):
    mesh = plsc.VectorSubcoreMesh(core_axis_name="c", subcore_axis_name="s")
    return pl.kernel(
        _sc_body,
        out_type=jax.ShapeDtypeStruct((NEXP, T, K), jnp.float32),
        mesh=mesh,
        compiler_params=pltpu.CompilerParams(needs_layout_passes=False),
        scratch_types=[
            pltpu.VMEM((CHT, EMBED), jnp.float32),
            pltpu.VMEM((CHT, EMBED), jnp.float32),
            pltpu.VMEM((CHT, K), jnp.float32),
            pltpu.VMEM((EMBED,), jnp.int32),
            pltpu.VMEM((EMBED,), jnp.int32),
            pltpu.VMEM((EMBED,), jnp.float32),
            pltpu.VMEM((EMBED,), jnp.float32),
            pltpu.VMEM((K,), jnp.float32),
            pltpu.VMEM((256,), jnp.int32),
            pltpu.VMEM((256,), jnp.int32),
            pltpu.VMEM((16,), jnp.float32),
        ],
    )(gate_all, h)


def kernel(hidden_states, posembed, W_pos, b_pos, W_gate, b_gate,
           Wg_e, Wu_e, Wd_e, Wg_s, Wu_s, Wd_s,
           ln_g, ln_b, W1, b1, W2, b2):
    h = hidden_states[0]
    b_pos2 = b_pos[None]
    b_gate2 = b_gate[None]

    pe = pl.pallas_call(
        _pe_body,
        out_shape=jax.ShapeDtypeStruct((NEXP, EMBED), jnp.float32),
    )(posembed, W_pos, b_pos2)

    grid = (T // TB,)
    full2 = lambda a, b: pl.BlockSpec((a, b), lambda i: (0, 0))
    tile2 = lambda w: pl.BlockSpec((TB, w), lambda i: (i, 0))

    gate_all = pl.pallas_call(
        _gate_body,
        grid=grid,
        in_specs=[tile2(EMBED), full2(NEXP, EMBED), full2(EMBED, EMBED),
                  full2(1, EMBED)],
        out_specs=pl.BlockSpec((NEXP, TB, EMBED), lambda i: (0, i, 0)),
        out_shape=jax.ShapeDtypeStruct((NEXP, T, EMBED), jnp.float32),
    )(h, pe, W_gate, b_gate2)

    xe = _sc_topk(gate_all, h)

    y0 = pl.pallas_call(
        _shared_body,
        grid=grid,
        in_specs=[tile2(EMBED), full2(INTER_S, EMBED), full2(INTER_S, EMBED),
                  full2(EMBED, INTER_S)],
        out_specs=tile2(EMBED),
        out_shape=jax.ShapeDtypeStruct((T, EMBED), jnp.float32),
    )(h, Wg_s, Wu_s, Wd_s)

    full3 = lambda s: pl.BlockSpec(s, lambda i: (0, 0, 0))
    out = pl.pallas_call(
        _post_body,
        grid=grid,
        in_specs=[pl.BlockSpec((NEXP, TB, K), lambda i: (0, i, 0)),
                  tile2(EMBED),
                  full3((NEXP, EMBED, K)), full3((NEXP, EMBED, K)),
                  full3((NEXP, EMBED, EMBED)),
                  full2(1, EMBED), full2(1, EMBED),
                  full2(EMBED, EMBED), full2(1, EMBED),
                  full2(EMBED, EMBED), full2(1, EMBED)],
        out_specs=tile2(EMBED),
        out_shape=jax.ShapeDtypeStruct((T, EMBED), jnp.float32),
    )(xe, y0, Wg_e, Wu_e, Wd_e, ln_g[None], ln_b[None],
      W1, b1[None], W2, b2[None])
    return out[None]


# speedup vs baseline: 1.1676x; 1.1676x over previous
import jax
import jax.numpy as jnp
from jax import lax
from jax.experimental import pallas as pl
from jax.experimental.pallas import tpu as pltpu, tpu_sc as plsc

EMBED = 768
NEXP = 8
K = 384
INTER_S = 1536
T = 2048
TB = 256
NW = 32
TOK_PER_W = T // NW
CHT = 8
NCHUNK = TOK_PER_W // CHT
NV = EMBED // 16
KV = K // 16
MASK31 = 0x7FFFFFFF

_CONTRACT_MINOR = (((1,), (1,)), ((), ()))


def _dotT(a, b):
    return lax.dot_general(a, b, _CONTRACT_MINOR,
                           preferred_element_type=jnp.float32)


def _pe_body(pos_ref, Wp_ref, bp_ref, pe_ref):
    logits = _dotT(pos_ref[...], Wp_ref[...]) + bp_ref[...]
    z = logits - jnp.max(logits, axis=-1, keepdims=True)
    e = jnp.exp(z)
    pe_ref[...] = e / jnp.sum(e, axis=-1, keepdims=True)


def _gate_body(h_ref, pe_ref, Wg_ref, bg_ref, gate_ref):
    h = h_ref[...]
    for e in range(NEXP):
        gate_ref[e] = _dotT(h * pe_ref[e], Wg_ref[...]) + bg_ref[...]


def _shared_body(h_ref, Wgs_ref, Wus_ref, Wds_ref, y0_ref):
    h = h_ref[...]
    m = jax.nn.silu(_dotT(h, Wgs_ref[...])) * _dotT(h, Wus_ref[...])
    y0_ref[...] = _dotT(m, Wds_ref[...])


def _post_body(xe_ref, y0_ref, Wge_ref, Wue_ref, Wde_ref,
               lng_ref, lnb_ref, W1_ref, b1_ref, W2_ref, b2_ref, o_ref):
    y = y0_ref[...]
    for e in range(NEXP):
        x = xe_ref[e]
        a = _dotT(x, Wge_ref[e])
        b = _dotT(x, Wue_ref[e])
        y = y + _dotT(jax.nn.silu(a) * b, Wde_ref[e])
    mean = jnp.mean(y, axis=-1, keepdims=True)
    var = jnp.mean((y - mean) ** 2, axis=-1, keepdims=True)
    y = (y - mean) * lax.rsqrt(var + 1e-6) * lng_ref[...] + lnb_ref[...]
    z = jax.nn.silu(_dotT(y, W1_ref[...]) + b1_ref[...])
    o_ref[...] = _dotT(z, W2_ref[...]) + b2_ref[...]


def _desc_key(f):
    u = plsc.bitcast(f, jnp.int32)
    s = lax.shift_right_arithmetic(u, 31)
    return jnp.bitwise_xor(u, jnp.bitwise_and(jnp.bitwise_not(s), MASK31))


def _inv_desc_key(k):
    s = lax.shift_right_arithmetic(k, 31)
    u = jnp.bitwise_xor(k, jnp.bitwise_and(jnp.bitwise_not(s), MASK31))
    return plsc.bitcast(u, jnp.float32)


def _digit(k, p):
    if p:
        k = lax.shift_right_logical(k, 8 * p)
    return jnp.bitwise_and(k, 255)


def _sc_body(gate_hbm, h_hbm, xe_hbm,
             hbuf, gbuf, xbuf, kA, kB, pA, pB, ebuf,
             hist0, hist1, hist2, hist3, offs, sbuf, dbuf, cbuf, mbuf):
    hists = (hist0, hist1, hist2, hist3)
    wid = lax.axis_index("s") * 2 + lax.axis_index("c")
    zero16 = jnp.zeros((16,), jnp.int32)
    full15 = jnp.full((16,), 15, jnp.int32)

    def scan_offsets(p):
        run = jnp.int32(0)
        for v in range(16):
            sl = pl.ds(v * 16, 16)
            hv = hists[p][sl]
            inc = plsc.cumsum(hv)
            offs[sl] = inc - hv + run
            run = run + jnp.sum(hv)

    def radix_pass(p, r):
        ksrc = kA if p % 2 == 0 else kB
        kdst = kB if p % 2 == 0 else kA
        pdst = pB if p % 2 == 0 else pA
        if p == 0:
            def psrc(sl):
                return hbuf[r, sl]
        elif p % 2 == 1:
            def psrc(sl):
                return pB[sl]
        else:
            def psrc(sl):
                return pA[sl]
        scan_offsets(p)
        if p > 0:
            for v in range(NV):
                sl = pl.ds(v * 16, 16)
                d = _digit(ksrc[sl], p)
                cnt, lastm = plsc.scan_count(d)
                dbuf[sl] = d
                cbuf[sl] = cnt
                mbuf[sl] = jnp.where(lastm, 1, 0)
        for v in range(NV):
            sl = pl.ds(v * 16, 16)
            d = dbuf[sl]
            cnt = cbuf[sl]
            lastm = mbuf[sl] == 1
            base = plsc.load_gather(offs, [d])
            pos = base + cnt - 1
            plsc.store_scatter(kdst, [pos], ksrc[sl])
            plsc.store_scatter(pdst, [pos], psrc(sl))
            plsc.addupdate_scatter(offs, [d], cnt, mask=lastm)

    def row_body(r, carry):
        for v in range(8):
            sl = pl.ds(v * 16, 16)
            hist0[sl] = zero16
            hist1[sl] = zero16
            hist2[sl] = zero16
            hist3[sl] = zero16
            sl2 = pl.ds(128 + v * 16, 16)
            hist0[sl2] = zero16
            hist1[sl2] = zero16
            hist2[sl2] = zero16
            hist3[sl2] = zero16
        for v in range(NV):
            sl = pl.ds(v * 16, 16)
            k = _desc_key(gbuf[r, sl])
            kA[sl] = k
            d0 = _digit(k, 0)
            d1 = _digit(k, 1)
            d2 = _digit(k, 2)
            d3 = _digit(k, 3)
            cnt0, last0 = plsc.scan_count(d0)
            cnt1, last1 = plsc.scan_count(d1)
            cnt2, last2 = plsc.scan_count(d2)
            cnt3, last3 = plsc.scan_count(d3)
            plsc.addupdate_scatter(hist0, [d0], cnt0, mask=last0)
            plsc.addupdate_scatter(hist1, [d1], cnt1, mask=last1)
            plsc.addupdate_scatter(hist2, [d2], cnt2, mask=last2)
            plsc.addupdate_scatter(hist3, [d3], cnt3, mask=last3)
            dbuf[sl] = d0
            cbuf[sl] = cnt0
            mbuf[sl] = jnp.where(last0, 1, 0)
        for p in range(4):
            radix_pass(p, r)
        acc = jnp.zeros((16,), jnp.float32)
        v0 = _inv_desc_key(kA[pl.ds(0, 16)])
        sbuf[...] = plsc.cummax(v0)
        mv = plsc.load_gather(sbuf, [full15])
        for j in range(KV):
            sl = pl.ds(j * 16, 16)
            ej = jnp.exp(_inv_desc_key(kA[sl]) - mv)
            ebuf[sl] = ej
            acc = acc + ej
        sbuf[...] = plsc.cumsum(acc)
        tot = plsc.load_gather(sbuf, [full15])
        winv = 1.0 / tot
        for j in range(KV):
            sl = pl.ds(j * 16, 16)
            xbuf[r, sl] = ebuf[sl] * winv * pA[sl]
        return carry

    def exp_body(e, carry, t0):
        pltpu.sync_copy(gate_hbm.at[e, pl.ds(t0, CHT)], gbuf)
        lax.fori_loop(0, CHT, row_body, 0)
        pltpu.sync_copy(xbuf, xe_hbm.at[e, pl.ds(t0, CHT)])
        return carry

    def chunk_body(ci, carry):
        t0 = wid * TOK_PER_W + ci * CHT
        pltpu.sync_copy(h_hbm.at[pl.ds(t0, CHT)], hbuf)
        lax.fori_loop(0, NEXP, lambda e, c: exp_body(e, c, t0), 0)
        return carry

    lax.fori_loop(0, NCHUNK, chunk_body, 0)


def _sc_topk(gate_all, h):
    mesh = plsc.VectorSubcoreMesh(core_axis_name="c", subcore_axis_name="s")
    return pl.kernel(
        _sc_body,
        out_type=jax.ShapeDtypeStruct((NEXP, T, K), jnp.float32),
        mesh=mesh,
        compiler_params=pltpu.CompilerParams(needs_layout_passes=False),
        scratch_types=[
            pltpu.VMEM((CHT, EMBED), jnp.float32),
            pltpu.VMEM((CHT, EMBED), jnp.float32),
            pltpu.VMEM((CHT, K), jnp.float32),
            pltpu.VMEM((EMBED,), jnp.int32),
            pltpu.VMEM((EMBED,), jnp.int32),
            pltpu.VMEM((EMBED,), jnp.float32),
            pltpu.VMEM((EMBED,), jnp.float32),
            pltpu.VMEM((K,), jnp.float32),
            pltpu.VMEM((256,), jnp.int32),
            pltpu.VMEM((256,), jnp.int32),
            pltpu.VMEM((256,), jnp.int32),
            pltpu.VMEM((256,), jnp.int32),
            pltpu.VMEM((256,), jnp.int32),
            pltpu.VMEM((16,), jnp.float32),
            pltpu.VMEM((EMBED,), jnp.int32),
            pltpu.VMEM((EMBED,), jnp.int32),
            pltpu.VMEM((EMBED,), jnp.int32),
        ],
    )(gate_all, h)


def kernel(hidden_states, posembed, W_pos, b_pos, W_gate, b_gate,
           Wg_e, Wu_e, Wd_e, Wg_s, Wu_s, Wd_s,
           ln_g, ln_b, W1, b1, W2, b2):
    h = hidden_states[0]
    b_pos2 = b_pos[None]
    b_gate2 = b_gate[None]

    pe = pl.pallas_call(
        _pe_body,
        out_shape=jax.ShapeDtypeStruct((NEXP, EMBED), jnp.float32),
    )(posembed, W_pos, b_pos2)

    grid = (T // TB,)
    full2 = lambda a, b: pl.BlockSpec((a, b), lambda i: (0, 0))
    tile2 = lambda w: pl.BlockSpec((TB, w), lambda i: (i, 0))

    gate_all = pl.pallas_call(
        _gate_body,
        grid=grid,
        in_specs=[tile2(EMBED), full2(NEXP, EMBED), full2(EMBED, EMBED),
                  full2(1, EMBED)],
        out_specs=pl.BlockSpec((NEXP, TB, EMBED), lambda i: (0, i, 0)),
        out_shape=jax.ShapeDtypeStruct((NEXP, T, EMBED), jnp.float32),
    )(h, pe, W_gate, b_gate2)

    xe = _sc_topk(gate_all, h)

    y0 = pl.pallas_call(
        _shared_body,
        grid=grid,
        in_specs=[tile2(EMBED), full2(INTER_S, EMBED), full2(INTER_S, EMBED),
                  full2(EMBED, INTER_S)],
        out_specs=tile2(EMBED),
        out_shape=jax.ShapeDtypeStruct((T, EMBED), jnp.float32),
    )(h, Wg_s, Wu_s, Wd_s)

    full3 = lambda s: pl.BlockSpec(s, lambda i: (0, 0, 0))
    out = pl.pallas_call(
        _post_body,
        grid=grid,
        in_specs=[pl.BlockSpec((NEXP, TB, K), lambda i: (0, i, 0)),
                  tile2(EMBED),
                  full3((NEXP, EMBED, K)), full3((NEXP, EMBED, K)),
                  full3((NEXP, EMBED, EMBED)),
                  full2(1, EMBED), full2(1, EMBED),
                  full2(EMBED, EMBED), full2(1, EMBED),
                  full2(EMBED, EMBED), full2(1, EMBED)],
        out_specs=tile2(EMBED),
        out_shape=jax.ShapeDtypeStruct((T, EMBED), jnp.float32),
    )(xe, y0, Wg_e, Wu_e, Wd_e, ln_g[None], ln_b[None],
      W1, b1[None], W2, b2[None])
    return out[None]
